# baseline (speedup 1.0000x reference)
import functools

import jax
import jax.numpy as jnp
from jax import lax
from jax.experimental import pallas as pl
from jax.experimental.pallas import tpu as pltpu
from jax.experimental.pallas import tpu_sc as plsc

_OFFSET = 2


def kernel(input_ids, pos_table):
    bsz, seq_len = input_ids.shape
    dim = pos_table.shape[-1]

    info = plsc.get_sparse_core_info()
    num_cores, num_subcores = info.num_cores, info.num_subcores
    num_lanes = info.num_lanes
    num_workers = num_cores * num_subcores
    rows_per_worker = seq_len // num_workers
    chunk_rows = 32
    n_chunks = rows_per_worker // chunk_rows

    read_rows = chunk_rows + 8

    @functools.partial(
        pl.kernel,
        mesh=plsc.VectorSubcoreMesh(core_axis_name="c", subcore_axis_name="s"),
        out_type=jax.ShapeDtypeStruct((bsz, seq_len, dim), jnp.float32),
        scratch_types=[
            pltpu.VMEM((read_rows, dim), jnp.float32),
            pltpu.VMEM((read_rows, dim), jnp.float32),
            pltpu.VMEM((read_rows, dim), jnp.float32),
            pltpu.SemaphoreType.DMA,
            pltpu.SemaphoreType.DMA,
        ],
    )
    def pe_kernel(table_hbm, out_hbm, buf0, buf1, buf2, in_sem, out_sem):
        wid = lax.axis_index("s") * num_cores + lax.axis_index("c")
        base = wid * rows_per_worker
        bufs = (buf0, buf1, buf2)

        def read(g, buf):
            row0 = base + g * chunk_rows
            return pltpu.async_copy(
                table_hbm.at[pl.ds(row0, read_rows), :], buf, in_sem
            )

        def shift_and_write(g, buf):
            row0 = base + g * chunk_rows
            n_j = dim // (num_lanes * 8)
            copies = []
            for k in range(chunk_rows // 8):
                r0 = k * 8

                def col_body(j, _, r0=r0):
                    c0 = j * (num_lanes * 8)
                    for jj in range(8):
                        c = c0 + jj * num_lanes
                        for i in range(8):
                            buf[r0 + i, pl.ds(c, num_lanes)] = buf[
                                r0 + i + _OFFSET, pl.ds(c, num_lanes)
                            ]
                    return 0

                lax.fori_loop(0, n_j, col_body, 0)
                for b in range(bsz):
                    copies.append(
                        pltpu.async_copy(
                            buf.at[pl.ds(r0, 8), :],
                            out_hbm.at[b, pl.ds(row0 + r0, 8), :],
                            out_sem,
                        )
                    )
            return copies

        rds = {0: read(0, bufs[0])}
        if n_chunks > 1:
            rds[1] = read(1, bufs[1])
        pending = {}
        for g in range(n_chunks):
            rds[g].wait()
            pending[g] = shift_and_write(g, bufs[g % 3])
            if g - 1 in pending:
                for c in pending.pop(g - 1):
                    c.wait()
            if g + 2 < n_chunks:
                rds[g + 2] = read(g + 2, bufs[(g + 2) % 3])
        for copies in pending.values():
            for c in copies:
                c.wait()

    return pe_kernel(pos_table)


# speedup vs baseline: 1.1119x; 1.1119x over previous
import functools

import jax
import jax.numpy as jnp
from jax import lax
from jax.experimental import pallas as pl
from jax.experimental.pallas import tpu as pltpu
from jax.experimental.pallas import tpu_sc as plsc

_OFFSET = 2


def kernel(input_ids, pos_table):
    bsz, seq_len = input_ids.shape
    dim = pos_table.shape[-1]

    info = plsc.get_sparse_core_info()
    num_cores, num_subcores = info.num_cores, info.num_subcores
    num_lanes = info.num_lanes
    num_workers = num_cores * num_subcores
    rows_per_worker = seq_len // num_workers
    chunk_rows = 32
    n_chunks = rows_per_worker // chunk_rows


    @functools.partial(
        pl.kernel,
        mesh=plsc.VectorSubcoreMesh(core_axis_name="c", subcore_axis_name="s"),
        out_type=jax.ShapeDtypeStruct((bsz, seq_len, dim), jnp.float32),
        scratch_types=[
            pltpu.VMEM((chunk_rows, dim), jnp.float32),
            pltpu.VMEM((chunk_rows, dim), jnp.float32),
            pltpu.VMEM((chunk_rows, dim), jnp.float32),
            pltpu.VMEM((8, dim), jnp.float32),
            pltpu.SemaphoreType.DMA,
            pltpu.SemaphoreType.DMA,
            pltpu.SemaphoreType.DMA,
        ],
    )
    def pe_kernel(
        table_hbm, out_hbm, buf0, buf1, buf2, side, in_sem, side_sem, out_sem
    ):
        wid = lax.axis_index("s") * num_cores + lax.axis_index("c")
        base = wid * rows_per_worker
        bufs = (buf0, buf1, buf2)

        def read(g, buf):
            row0 = base + g * chunk_rows
            return pltpu.async_copy(
                table_hbm.at[pl.ds(row0, chunk_rows), :], buf, in_sem
            )

        n_j = dim // (num_lanes * 8)

        def shift_rows(dst, dst_r0, src, src_r0, n_rows):
            def col_body(j, _):
                c0 = j * (num_lanes * 8)
                for jj in range(8):
                    c = c0 + jj * num_lanes
                    for i in range(n_rows):
                        dst[dst_r0 + i, pl.ds(c, num_lanes)] = src[
                            src_r0 + i, pl.ds(c, num_lanes)
                        ]
                return 0

            lax.fori_loop(0, n_j, col_body, 0)

        def write(g, buf, r0, n_rows):
            row0 = base + g * chunk_rows
            return [
                pltpu.async_copy(
                    buf.at[pl.ds(r0, n_rows), :],
                    out_hbm.at[b, pl.ds(row0 + r0, n_rows), :],
                    out_sem,
                )
                for b in range(bsz)
            ]

        rds = {0: read(0, bufs[0])}
        if n_chunks > 1:
            rds[1] = read(1, bufs[1])
        side_rd = pltpu.async_copy(
            table_hbm.at[pl.ds(base + rows_per_worker, 8), :], side, side_sem
        )
        pending = {}
        for g in range(n_chunks):
            buf = bufs[g % 3]
            if g == 0:
                rds[0].wait()
            for k in range(chunk_rows // 8 - 1):
                shift_rows(buf, k * 8, buf, k * 8 + _OFFSET, 8)
            bulk = write(g, buf, 0, chunk_rows - 8)
            shift_rows(
                buf, chunk_rows - 8, buf, chunk_rows - 8 + _OFFSET, 8 - _OFFSET
            )
            if g + 1 < n_chunks:
                rds[g + 1].wait()
                nxt = bufs[(g + 1) % 3]
            else:
                side_rd.wait()
                nxt = side
            shift_rows(buf, chunk_rows - _OFFSET, nxt, 0, _OFFSET)
            pending[g] = bulk + write(g, buf, chunk_rows - 8, 8)
            if g - 1 in pending:
                for c in pending.pop(g - 1):
                    c.wait()
            if g + 2 < n_chunks:
                rds[g + 2] = read(g + 2, bufs[(g + 2) % 3])
        for copies in pending.values():
            for c in copies:
                c.wait()

    return pe_kernel(pos_table)
